# Initial kernel scaffold; baseline (speedup 1.0000x reference)
import jax
import jax.numpy as jnp
from jax.experimental import pallas as pl


def kernel(token_ids, tid2eid):
    raise NotImplementedError("write your pallas kernel here")


# speedup vs baseline: 4.4296x; 4.4296x over previous
import functools

import jax
import jax.numpy as jnp
from jax import lax
from jax.experimental import pallas as pl
from jax.experimental.pallas import tpu as pltpu
from jax.experimental.pallas import tpu_sc as plsc

NUM_EXPERTS = 64
TOPK = 2


_GCHUNK = 128
_LANES = 16


def _sc_gather(table_flat, flat_ids):
    n = flat_ids.shape[0]
    info = plsc.get_sparse_core_info()
    num_workers = info.num_cores * info.num_subcores
    b_per_w = n // num_workers
    n_chunks = b_per_w // _GCHUNK
    ids3d = flat_ids.reshape(num_workers, n_chunks, _GCHUNK)
    mesh = plsc.VectorSubcoreMesh(core_axis_name="c", subcore_axis_name="s")

    @functools.partial(
        pl.kernel,
        mesh=mesh,
        compiler_params=pltpu.CompilerParams(use_tc_tiling_on_sc=False),
        out_type=(
            jax.ShapeDtypeStruct((n,), jnp.int32),
            jax.ShapeDtypeStruct((n,), jnp.int32),
        ),
        scratch_types=[
            pltpu.VMEM((n_chunks, _GCHUNK), jnp.int32),
            pltpu.VMEM((n_chunks, _GCHUNK), jnp.int32),
            pltpu.VMEM((b_per_w,), jnp.int32),
            pltpu.VMEM((b_per_w,), jnp.int32),
            pltpu.SemaphoreType.DMA,
        ],
    )
    def gather_kernel(table_hbm, ids_hbm, e0_hbm, e1_hbm, idx_v, sidx_v,
                      e0_v, e1_v, sem):
        wid = lax.axis_index("s") * info.num_cores + lax.axis_index("c")
        base = wid * b_per_w
        pltpu.sync_copy(ids_hbm.at[wid], idx_v)
        for j in range(n_chunks):
            for r in range(_GCHUNK // _LANES):
                v = idx_v[j, pl.ds(r * _LANES, _LANES)]
                sidx_v[j, pl.ds(r * _LANES, _LANES)] = v + v
        for j in range(n_chunks):
            pltpu.async_copy(
                table_hbm.at[sidx_v.at[j]],
                e0_v.at[pl.ds(j * _GCHUNK, _GCHUNK)],
                sem,
            )
        for j in range(n_chunks):
            pltpu.make_async_copy(
                table_hbm.at[sidx_v.at[j]],
                e0_v.at[pl.ds(j * _GCHUNK, _GCHUNK)],
                sem,
            ).wait()
        pltpu.sync_copy(e0_v, e0_hbm.at[pl.ds(base, b_per_w)])
        for j in range(n_chunks):
            for r in range(_GCHUNK // _LANES):
                sl = pl.ds(r * _LANES, _LANES)
                sidx_v[j, sl] = sidx_v[j, sl] + 1
        for j in range(n_chunks):
            pltpu.async_copy(
                table_hbm.at[sidx_v.at[j]],
                e1_v.at[pl.ds(j * _GCHUNK, _GCHUNK)],
                sem,
            )
        for j in range(n_chunks):
            pltpu.make_async_copy(
                table_hbm.at[sidx_v.at[j]],
                e1_v.at[pl.ds(j * _GCHUNK, _GCHUNK)],
                sem,
            ).wait()
        pltpu.sync_copy(e1_v, e1_hbm.at[pl.ds(base, b_per_w)])

    return gather_kernel(table_flat, ids3d)


def _tc_expand(e0, e1):
    n = e0.shape[0]
    block = 2048

    def body(e0_ref, e1_ref, probs_ref, map_ref):
        iota = lax.broadcasted_iota(jnp.int32, (block, NUM_EXPERTS), 1)
        hit = (iota == e0_ref[...]) | (iota == e1_ref[...])
        probs_ref[...] = jnp.where(hit, jnp.float32(1.0 / TOPK), jnp.float32(0.0))
        map_ref[...] = hit

    return pl.pallas_call(
        body,
        grid=(n // block,),
        in_specs=[
            pl.BlockSpec((block, 1), lambda i: (i, 0)),
            pl.BlockSpec((block, 1), lambda i: (i, 0)),
        ],
        out_specs=[
            pl.BlockSpec((block, NUM_EXPERTS), lambda i: (i, 0)),
            pl.BlockSpec((block, NUM_EXPERTS), lambda i: (i, 0)),
        ],
        out_shape=[
            jax.ShapeDtypeStruct((n, NUM_EXPERTS), jnp.float32),
            jax.ShapeDtypeStruct((n, NUM_EXPERTS), jnp.bool_),
        ],
    )(e0, e1)


def kernel(token_ids, tid2eid):
    flat_ids = token_ids.reshape(-1)
    e0, e1 = _sc_gather(tid2eid.reshape(-1), flat_ids)
    probs, routing_map = _tc_expand(e0[:, None], e1[:, None])
    return probs, routing_map


# speedup vs baseline: 16.1662x; 3.6496x over previous
import functools

import jax
import jax.numpy as jnp
from jax import lax
from jax.experimental import pallas as pl
from jax.experimental.pallas import tpu as pltpu
from jax.experimental.pallas import tpu_sc as plsc

NUM_EXPERTS = 64
TOPK = 2
_GCHUNK = 128


def _sc_gather(t0, t1, flat_ids):
    n = flat_ids.shape[0]
    info = plsc.get_sparse_core_info()
    num_workers = info.num_cores * info.num_subcores
    b_per_w = n // num_workers
    n_chunks = b_per_w // _GCHUNK
    ids3d = flat_ids.reshape(num_workers, n_chunks, _GCHUNK)
    mesh = plsc.VectorSubcoreMesh(core_axis_name="c", subcore_axis_name="s")

    @functools.partial(
        pl.kernel,
        mesh=mesh,
        compiler_params=pltpu.CompilerParams(use_tc_tiling_on_sc=False),
        out_type=(
            jax.ShapeDtypeStruct((n,), jnp.int32),
            jax.ShapeDtypeStruct((n,), jnp.int32),
        ),
        scratch_types=[
            pltpu.VMEM((n_chunks, _GCHUNK), jnp.int32),
            pltpu.VMEM((b_per_w,), jnp.int32),
            pltpu.VMEM((b_per_w,), jnp.int32),
            pltpu.SemaphoreType.DMA,
        ],
    )
    def gather_kernel(t0_hbm, t1_hbm, ids_hbm, e0_hbm, e1_hbm,
                      idx_v, e0_v, e1_v, sem):
        wid = lax.axis_index("s") * info.num_cores + lax.axis_index("c")
        base = wid * b_per_w
        pltpu.sync_copy(ids_hbm.at[wid], idx_v)
        for j in range(n_chunks):
            pltpu.async_copy(
                t0_hbm.at[idx_v.at[j]],
                e0_v.at[pl.ds(j * _GCHUNK, _GCHUNK)],
                sem,
            )
            pltpu.async_copy(
                t1_hbm.at[idx_v.at[j]],
                e1_v.at[pl.ds(j * _GCHUNK, _GCHUNK)],
                sem,
            )
        for j in range(n_chunks):
            pltpu.make_async_copy(
                t0_hbm.at[idx_v.at[j]],
                e0_v.at[pl.ds(j * _GCHUNK, _GCHUNK)],
                sem,
            ).wait()
            pltpu.make_async_copy(
                t1_hbm.at[idx_v.at[j]],
                e1_v.at[pl.ds(j * _GCHUNK, _GCHUNK)],
                sem,
            ).wait()
        pltpu.sync_copy(e0_v, e0_hbm.at[pl.ds(base, b_per_w)])
        pltpu.sync_copy(e1_v, e1_hbm.at[pl.ds(base, b_per_w)])

    return gather_kernel(t0, t1, ids3d)


def _tc_expand(e0, e1):
    g, _, b = e0.shape
    n = g * b

    def body(e0_ref, e1_ref, probs_ref, map_ref):
        iota = lax.broadcasted_iota(jnp.int32, (NUM_EXPERTS, b), 0)
        ee0 = jnp.broadcast_to(e0_ref[0], (NUM_EXPERTS, b))
        ee1 = jnp.broadcast_to(e1_ref[0], (NUM_EXPERTS, b))
        hit = (iota == ee0) | (iota == ee1)
        probs_ref[...] = jnp.where(hit, jnp.float32(1.0 / TOPK), jnp.float32(0.0))
        map_ref[...] = hit.astype(jnp.int8)

    return pl.pallas_call(
        body,
        grid=(g,),
        in_specs=[
            pl.BlockSpec((1, 1, b), lambda i: (i, 0, 0)),
            pl.BlockSpec((1, 1, b), lambda i: (i, 0, 0)),
        ],
        out_specs=[
            pl.BlockSpec((NUM_EXPERTS, b), lambda i: (0, i)),
            pl.BlockSpec((NUM_EXPERTS, b), lambda i: (0, i)),
        ],
        out_shape=[
            jax.ShapeDtypeStruct((NUM_EXPERTS, n), jnp.float32),
            jax.ShapeDtypeStruct((NUM_EXPERTS, n), jnp.int8),
        ],
    )(e0, e1)


_TC_BLOCK = 2048


def kernel(token_ids, tid2eid):
    flat_ids = token_ids.reshape(-1)
    n = flat_ids.shape[0]
    t0 = tid2eid[:, 0]
    t1 = tid2eid[:, 1]
    e0, e1 = _sc_gather(t0, t1, flat_ids)
    g = n // _TC_BLOCK
    probs_t, map_t = _tc_expand(
        e0.reshape(g, 1, _TC_BLOCK), e1.reshape(g, 1, _TC_BLOCK)
    )
    return probs_t.T, map_t.T.astype(bool)
